# baseline (speedup 1.0000x reference)
import dataclasses
import functools

import jax
import jax.numpy as jnp
from jax import lax
from jax.experimental import pallas as pl
from jax.experimental.pallas import tpu as pltpu
from jax.experimental.pallas import tpu_sc as plsc

NUM_SEG = 64
D = 512
N = 50000
LANES = 16
NC = 2
NS = 16
NW = NC * NS
NVREG = D // LANES

B = 16
NBLK = N // B
BPT = 104

R_TC = 2000
GRID_TC = N // R_TC
BLK_TC = R_TC // B
GRID_PAD = -(-NW * BPT // BLK_TC)
NB_PAD = NW * BPT * LANES

_mesh = plsc.VectorSubcoreMesh(core_axis_name="c", subcore_axis_name="s")

_sc_params = pltpu.CompilerParams()
if "needs_layout_passes" in pltpu.CompilerParams.__dataclass_fields__:
    _sc_params = dataclasses.replace(_sc_params, needs_layout_passes=False)


def _tc_block_body(x_ref, p_ref):
    xr = x_ref[...]
    p_ref[0] = jnp.sum(xr.reshape(BLK_TC, B, D), axis=1)


def _tc_block_sums(x):
    p3 = pl.pallas_call(
        _tc_block_body,
        grid=(GRID_TC,),
        in_specs=[pl.BlockSpec((R_TC, D), lambda i: (i, 0))],
        out_specs=pl.BlockSpec((1, BLK_TC, D), lambda i: (i, 0, 0)),
        out_shape=jax.ShapeDtypeStruct((GRID_PAD, BLK_TC, D), jnp.float32),
        compiler_params=pltpu.CompilerParams(
            dimension_semantics=("parallel",)),
    )(x)
    return p3.reshape(GRID_PAD * BLK_TC, D)


@functools.partial(
    pl.kernel,
    mesh=_mesh,
    compiler_params=_sc_params,
    out_type=(
        jax.ShapeDtypeStruct((NW, NUM_SEG, D), jnp.float32),
        jax.ShapeDtypeStruct((NW, NUM_SEG, LANES), jnp.float32),
    ),
    scratch_types=[
        pltpu.VMEM((BPT, D), jnp.float32),
        pltpu.VMEM((BPT * LANES,), jnp.int32),
        pltpu.VMEM((LANES, D), jnp.float32),
        pltpu.VMEM((NUM_SEG, D), jnp.float32),
        pltpu.VMEM((NUM_SEG, LANES), jnp.float32),
        pltpu.SemaphoreType.DMA,
        pltpu.SemaphoreType.DMA,
        pltpu.SemaphoreType.DMA,
        pltpu.SemaphoreType.DMA,
    ],
)
def _sc_seg_sum(p_hbm, b_hbm, x_hbm, psum_hbm, pcnt_hbm,
                pchunk, bchunk, xrow, acc, cnt, semp, semb, semx, semo):
    wid = lax.axis_index("s") * NC + lax.axis_index("c")
    blk0 = wid * BPT
    nblk = jnp.maximum(0, jnp.minimum(BPT, NBLK - blk0))

    zeros16 = jnp.zeros((LANES,), jnp.float32)
    ones16 = jnp.ones((LANES,), jnp.float32)
    sixteen16 = jnp.full((LANES,), float(B), jnp.float32)

    pltpu.make_async_copy(
        p_hbm.at[pl.ds(blk0, BPT)], pchunk, semp).start()
    pltpu.make_async_copy(
        b_hbm.at[pl.ds(blk0 * LANES, BPT * LANES)], bchunk, semb).start()

    @pl.loop(0, NUM_SEG)
    def _zero(r):
        for j in range(NVREG):
            acc[r, pl.ds(j * LANES, LANES)] = zeros16
        cnt[r, pl.ds(0, LANES)] = zeros16

    pltpu.make_async_copy(
        p_hbm.at[pl.ds(blk0, BPT)], pchunk, semp).wait()
    pltpu.make_async_copy(
        b_hbm.at[pl.ds(blk0 * LANES, BPT * LANES)], bchunk, semb).wait()

    @pl.loop(0, nblk)
    def _blocks(bi):
        svec = bchunk[pl.ds(bi * LANES, LANES)]
        sfirst = svec[0]
        slast = svec[LANES - 1]

        @pl.when(sfirst == slast)
        def _uniform():
            for j in range(NVREG):
                sl = pl.ds(j * LANES, LANES)
                plsc.addupdate(acc.at[sfirst, sl], pchunk[bi, sl])
            plsc.addupdate(cnt.at[sfirst, pl.ds(0, LANES)], sixteen16)

        @pl.when(sfirst != slast)
        def _boundary():
            g = blk0 + bi
            pltpu.make_async_copy(
                x_hbm.at[pl.ds(g * B, B)], xrow, semx).start()
            pltpu.make_async_copy(
                x_hbm.at[pl.ds(g * B, B)], xrow, semx).wait()
            for k in range(LANES):
                s = svec[k]
                for j in range(NVREG):
                    sl = pl.ds(j * LANES, LANES)
                    plsc.addupdate(acc.at[s, sl], xrow[k, sl])
                plsc.addupdate(cnt.at[s, pl.ds(0, LANES)], ones16)

    pltpu.make_async_copy(acc, psum_hbm.at[wid], semo).start()
    pltpu.make_async_copy(cnt, pcnt_hbm.at[wid], semo).start()
    pltpu.make_async_copy(acc, psum_hbm.at[wid], semo).wait()
    pltpu.make_async_copy(cnt, pcnt_hbm.at[wid], semo).wait()


def _combine_body(ps_ref, pc_ref, out_ref):
    sums = jnp.sum(ps_ref[...], axis=0)
    counts = jnp.sum(pc_ref[...], axis=0)[:, 0:1]
    out_ref[...] = sums / jnp.clip(counts, 1.0, None)


def _tc_combine(psum, pcnt):
    return pl.pallas_call(
        _combine_body,
        out_shape=jax.ShapeDtypeStruct((NUM_SEG, D), jnp.float32),
    )(psum, pcnt)


@jax.jit
def kernel(x, batch):
    batch32 = batch.astype(jnp.int32)
    bpad = jnp.pad(batch32, (0, NB_PAD - N))
    p = _tc_block_sums(x)
    psum, pcnt = _sc_seg_sum(p, bpad, x)
    return _tc_combine(psum, pcnt)
